# Initial kernel scaffold; baseline (speedup 1.0000x reference)
import jax
import jax.numpy as jnp
from jax.experimental import pallas as pl


def kernel(input_ids, table_0, table_1, table_2, table_3, table_4, table_5, table_6, table_7):
    raise NotImplementedError("write your pallas kernel here")


# speedup vs baseline: 9.8534x; 9.8534x over previous
import functools

import jax
import jax.numpy as jnp
from jax import lax
from jax.experimental import pallas as pl
from jax.experimental.pallas import tpu as pltpu
from jax.experimental.pallas import tpu_sc as plsc

_PRIMES = (31, 43, 59, 61, 73, 97, 103, 113)
_BUCKETS = 100000
_K = 8
_SLICE = 16
_BATCH, _SEQ = 4096, 50
_N = _BATCH * _SEQ
_NC, _NS, _L = 2, 16, 16
_NW = _NC * _NS
_TPW = _N // _NW
_CB = 128
_NB = _TPW // _CB

_mesh = plsc.VectorSubcoreMesh(core_axis_name="c", subcore_axis_name="s")


@functools.partial(
    pl.kernel,
    out_type=jax.ShapeDtypeStruct((_N * _K, _SLICE), jnp.float32),
    mesh=_mesh,
    compiler_params=pltpu.CompilerParams(use_tc_tiling_on_sc=False),
    scratch_types=[
        pltpu.VMEM((_CB,), jnp.int32),
        pltpu.VMEM((_K, _CB), jnp.int32),
        pltpu.VMEM((_K, _CB), jnp.int32),
        pltpu.VMEM((_K, _CB, _SLICE), jnp.float32),
        pltpu.SemaphoreType.DMA,
        pltpu.SemaphoreType.DMA,
    ],
)
def _emb(ids_hbm, t0, t1, t2, t3, t4, t5, t6, t7, out_hbm,
         ids_v, idx_v, didx_v, rows_v, gsem, ssem):
    tables = (t0, t1, t2, t3, t4, t5, t6, t7)
    wid = lax.axis_index("s") * _NC + lax.axis_index("c")
    base = wid * _TPW
    lane = lax.iota(jnp.int32, _L)
    inv = jnp.float32(1.0 / _BUCKETS)

    def block(b, carry):
        tb = base + b * _CB
        pltpu.sync_copy(ids_hbm.at[pl.ds(tb, _CB)], ids_v)
        for j in range(_CB // _L):
            x = ids_v[pl.ds(j * _L, _L)]
            xf = x.astype(jnp.float32)
            t8 = (tb + j * _L + lane) * _K
            for k in range(_K):
                p = _PRIMES[k]
                xp = x * p
                q = (xf * jnp.float32(p) * inv).astype(jnp.int32)
                r = xp - q * _BUCKETS
                r = jnp.where(r < 0, r + _BUCKETS, r)
                r = jnp.where(r >= _BUCKETS, r - _BUCKETS, r)
                idx_v[k, pl.ds(j * _L, _L)] = r
                didx_v[k, pl.ds(j * _L, _L)] = t8 + k
        gets = [pltpu.async_copy(tables[k].at[idx_v.at[k]], rows_v.at[k], gsem)
                for k in range(_K)]
        for c in gets:
            c.wait()
        puts = [pltpu.async_copy(rows_v.at[k], out_hbm.at[didx_v.at[k]], ssem)
                for k in range(_K)]
        for c in puts:
            c.wait()
        return carry

    lax.fori_loop(0, _NB, block, 0)


def kernel(input_ids, table_0, table_1, table_2, table_3, table_4,
           table_5, table_6, table_7):
    ids = input_ids.reshape(-1).astype(jnp.int32)
    out = _emb(ids, table_0, table_1, table_2, table_3,
               table_4, table_5, table_6, table_7)
    return out.reshape(_BATCH, _SEQ, _K * _SLICE)
